# Initial kernel scaffold; baseline (speedup 1.0000x reference)
import jax
import jax.numpy as jnp
from jax.experimental import pallas as pl


def kernel(x, w_gating):
    raise NotImplementedError("write your pallas kernel here")


# speedup vs baseline: 3.3905x; 3.3905x over previous
import jax
import jax.numpy as jnp
from jax.experimental import pallas as pl
from jax.experimental.pallas import tpu as pltpu

EPS_ = 1e-09
NGATES = 64
BT = 512


def _gating_block(x_ref, w_ref, o_ref):
    logits = jnp.dot(x_ref[...], w_ref[...], preferred_element_type=jnp.float32)
    m = jnp.max(logits, axis=-1, keepdims=True)
    e = jnp.exp(logits - m)
    s = jnp.sum(e, axis=-1, keepdims=True)
    p = e / s
    cols = jax.lax.broadcasted_iota(jnp.int32, p.shape, 1)
    v1 = jnp.max(p, axis=-1, keepdims=True)
    eq1 = p == v1
    i1 = jnp.min(jnp.where(eq1, cols, NGATES), axis=-1, keepdims=True)
    mask1 = eq1 & (cols == i1)
    p2 = jnp.where(mask1, 0.0, p)
    v2 = jnp.max(p2, axis=-1, keepdims=True)
    eq2 = p2 == v2
    i2 = jnp.min(jnp.where(eq2, cols, NGATES), axis=-1, keepdims=True)
    mask2 = eq2 & (cols == i2)
    denom = v1 + v2 + EPS_
    out = jnp.where(mask2, v2 / denom, jnp.where(mask1, v1 / denom, 0.0))
    o_ref[...] = out


def kernel(x, w_gating):
    b, group, dim = x.shape
    n = b * group
    x2 = x.reshape(n, dim)
    grid = (n // BT,)
    out = pl.pallas_call(
        _gating_block,
        grid=grid,
        in_specs=[
            pl.BlockSpec((BT, dim), lambda i: (i, 0)),
            pl.BlockSpec((dim, NGATES), lambda i: (0, 0)),
        ],
        out_specs=pl.BlockSpec((BT, NGATES), lambda i: (i, 0)),
        out_shape=jax.ShapeDtypeStruct((n, NGATES), jnp.float32),
        compiler_params=pltpu.CompilerParams(
            dimension_semantics=("arbitrary",),
        ),
    )(x2, w_gating)
    return out.reshape(b, group, NGATES)


# speedup vs baseline: 3.6508x; 1.0768x over previous
import jax
import jax.numpy as jnp
from jax.experimental import pallas as pl
from jax.experimental.pallas import tpu as pltpu

EPS_ = 1e-09
NGATES = 64
BT = 512


def _gating_block(x_ref, w_ref, o_ref):
    logits = jnp.dot(x_ref[...], w_ref[...], preferred_element_type=jnp.float32)
    m = jnp.max(logits, axis=-1, keepdims=True)
    e = jnp.exp(logits - m)
    s = jnp.sum(e, axis=-1, keepdims=True)
    eq1 = e == 1.0
    e2 = jnp.where(eq1, 0.0, e)
    em2 = jnp.max(e2, axis=-1, keepdims=True)
    v1 = 1.0 / s
    v2 = em2 / s
    denom = v1 + v2 + EPS_
    eq2 = (e2 == em2) & ~eq1
    out = jnp.where(eq1, v1 / denom, 0.0)
    out = jnp.where(eq2, v2 / denom, out)
    cols0 = jax.lax.broadcasted_iota(jnp.int32, out.shape, 1) == 0
    out = jnp.where(cols0 & (v2 == 0.0), 0.0, out)
    o_ref[...] = out


def kernel(x, w_gating):
    b, group, dim = x.shape
    n = b * group
    x2 = x.reshape(n, dim)
    grid = (n // BT,)
    out = pl.pallas_call(
        _gating_block,
        grid=grid,
        in_specs=[
            pl.BlockSpec((BT, dim), lambda i: (i, 0)),
            pl.BlockSpec((dim, NGATES), lambda i: (0, 0)),
        ],
        out_specs=pl.BlockSpec((BT, NGATES), lambda i: (i, 0)),
        out_shape=jax.ShapeDtypeStruct((n, NGATES), jnp.float32),
        compiler_params=pltpu.CompilerParams(
            dimension_semantics=("arbitrary",),
        ),
    )(x2, w_gating)
    return out.reshape(b, group, NGATES)


# speedup vs baseline: 3.6907x; 1.0109x over previous
import jax
import jax.numpy as jnp
from jax.experimental import pallas as pl
from jax.experimental.pallas import tpu as pltpu

EPS_ = 1e-09
NGATES = 64
BT = 1024


def _gating_block(x_ref, w_ref, o_ref):
    logits = jnp.dot(x_ref[...], w_ref[...], preferred_element_type=jnp.float32)
    m = jnp.max(logits, axis=-1, keepdims=True)
    e = jnp.exp(logits - m)
    s = jnp.sum(e, axis=-1, keepdims=True)
    eq1 = e == 1.0
    e2 = jnp.where(eq1, 0.0, e)
    em2 = jnp.max(e2, axis=-1, keepdims=True)
    v1 = 1.0 / s
    v2 = em2 / s
    denom = v1 + v2 + EPS_
    eq2 = (e2 == em2) & ~eq1
    out = jnp.where(eq1, v1 / denom, 0.0)
    out = jnp.where(eq2, v2 / denom, out)
    cols0 = jax.lax.broadcasted_iota(jnp.int32, out.shape, 1) == 0
    out = jnp.where(cols0 & (v2 == 0.0), 0.0, out)
    o_ref[...] = out


def kernel(x, w_gating):
    b, group, dim = x.shape
    n = b * group
    x2 = x.reshape(n, dim)
    grid = (n // BT,)
    out = pl.pallas_call(
        _gating_block,
        grid=grid,
        in_specs=[
            pl.BlockSpec((BT, dim), lambda i: (i, 0)),
            pl.BlockSpec((dim, NGATES), lambda i: (0, 0)),
        ],
        out_specs=pl.BlockSpec((BT, NGATES), lambda i: (i, 0)),
        out_shape=jax.ShapeDtypeStruct((n, NGATES), jnp.float32),
        compiler_params=pltpu.CompilerParams(
            dimension_semantics=("arbitrary",),
        ),
    )(x2, w_gating)
    return out.reshape(b, group, NGATES)


# speedup vs baseline: 3.6926x; 1.0005x over previous
import jax
import jax.numpy as jnp
from jax.experimental import pallas as pl
from jax.experimental.pallas import tpu as pltpu

EPS_ = 1e-09
NGATES = 64
BT = 1024


def _gating_block(x_ref, w_ref, o_ref):
    logits = jnp.dot(x_ref[...], w_ref[...], preferred_element_type=jnp.float32)
    m = jnp.max(logits, axis=-1, keepdims=True)
    e = jnp.exp(logits - m)
    s = jnp.sum(e, axis=-1, keepdims=True)
    eq1 = e == 1.0
    e2 = jnp.where(eq1, 0.0, e)
    em2 = jnp.max(e2, axis=-1, keepdims=True)
    v1 = 1.0 / s
    v2 = em2 / s
    denom = v1 + v2 + EPS_
    eq2 = (e2 == em2) & ~eq1
    out = jnp.where(eq1, v1 / denom, 0.0)
    out = jnp.where(eq2, v2 / denom, out)
    cols0 = jax.lax.broadcasted_iota(jnp.int32, out.shape, 1) == 0
    out = jnp.where(cols0 & (v2 == 0.0), 0.0, out)
    o_ref[...] = out


def kernel(x, w_gating):
    b, group, dim = x.shape
    n = b * group
    x2 = x.reshape(n, dim)
    grid = (n // BT,)
    out = pl.pallas_call(
        _gating_block,
        grid=grid,
        in_specs=[
            pl.BlockSpec((BT, dim), lambda i: (i, 0)),
            pl.BlockSpec((dim, NGATES), lambda i: (0, 0)),
        ],
        out_specs=pl.BlockSpec((BT, NGATES), lambda i: (i, 0)),
        out_shape=jax.ShapeDtypeStruct((n, NGATES), jnp.float32),
        compiler_params=pltpu.CompilerParams(
            dimension_semantics=("parallel",),
        ),
    )(x2, w_gating)
    return out.reshape(b, group, NGATES)
